# Initial kernel scaffold; baseline (speedup 1.0000x reference)
import jax
import jax.numpy as jnp
from jax.experimental import pallas as pl


def kernel(outputs, labels):
    raise NotImplementedError("write your pallas kernel here")


# speedup vs baseline: 99.3137x; 99.3137x over previous
import functools

import jax
import jax.numpy as jnp
from jax import lax
from jax.experimental import pallas as pl
from jax.experimental.pallas import tpu as pltpu
from jax.experimental.pallas import tpu_sc as plsc

C = 21
K = 1024
HIST = C * K
B = 8
HW = 512 * 512
NW = 32
PIX_PER_W = (B * HW) // NW
CH = 2048
L = 16


def _sc_hist_kernel(x_hbm, lab_hbm, out_hbm, xbuf, lbuf, hist, sem):
    wid = lax.axis_index("c") * 16 + lax.axis_index("s")
    img = wid // 4
    base = (wid % 4) * PIX_PER_W

    zeros16 = jnp.zeros((L,), jnp.float32)

    def zero_body(i, carry):
        hist[pl.ds(i * L, L)] = zeros16
        return carry

    lax.fori_loop(0, (2 * HIST) // L, zero_body, 0)

    ones16 = jnp.ones((L,), jnp.float32)

    def chunk_body(t, carry):
        off = base + t * CH
        copies = [
            pltpu.make_async_copy(
                x_hbm.at[img, pl.ds(c, 1), pl.ds(off, CH)],
                xbuf.at[pl.ds(c, 1)], sem)
            for c in range(C)
        ]
        copies.append(
            pltpu.make_async_copy(lab_hbm.at[img, pl.ds(off, CH)], lbuf, sem))
        for cp in copies:
            cp.start()
        for cp in copies:
            cp.wait()

        def group_body(g, inner):
            s16 = pl.ds(g * L, L)
            labv = lbuf[s16]
            vs = [xbuf[c, s16] for c in range(C)]
            m = vs[0]
            for c in range(1, C):
                m = jnp.maximum(m, vs[c])
            es = [jnp.exp(vs[c] - m) for c in range(C)]
            ssum = es[0]
            for c in range(1, C):
                ssum = ssum + es[c]
            rinv = 1.0 / ssum
            for c in range(C):
                p = es[c] * rinv
                fg = labv == c
                err = jnp.where(fg, 1.0 - p, p)
                bin_ = jnp.minimum((err * K).astype(jnp.int32), K - 1)
                idx = bin_ + (c * K)
                plsc.addupdate_scatter(hist, [idx], ones16)
                plsc.addupdate_scatter(hist, [idx + HIST], ones16, mask=fg)
            return inner

        lax.fori_loop(0, CH // L, group_body, 0)
        return carry

    lax.fori_loop(0, PIX_PER_W // CH, chunk_body, 0)

    pltpu.sync_copy(hist, out_hbm.at[wid])


_sc_hist = functools.partial(
    pl.kernel,
    mesh=plsc.VectorSubcoreMesh(core_axis_name="c", subcore_axis_name="s"),
    out_type=jax.ShapeDtypeStruct((NW, 2 * HIST), jnp.float32),
    scratch_types=[
        pltpu.VMEM((C, CH), jnp.float32),
        pltpu.VMEM((CH,), jnp.int32),
        pltpu.VMEM((2 * HIST,), jnp.float32),
        pltpu.SemaphoreType.DMA,
    ],
    compiler_params=pltpu.CompilerParams(needs_layout_passes=False),
)(_sc_hist_kernel)


def _tc_finish_kernel(h_ref, out_ref):
    nf = jnp.sum(h_ref[...], axis=0)
    jj = lax.broadcasted_iota(jnp.int32, (K, K), 0)
    kk = lax.broadcasted_iota(jnp.int32, (K, K), 1)
    tri = (jj >= kk).astype(jnp.float32)
    cum = jnp.dot(nf, tri, preferred_element_type=jnp.float32)
    n, f = nf[:C], nf[C:]
    Ninc, Finc = cum[:C], cum[C:]
    G = Finc[:, 0:1]

    def jac(N, F):
        den = G + N - F
        return jnp.where(den > 0, 1.0 - (G - F) / jnp.maximum(den, 1.0), 0.0)

    dJ = jac(Ninc, Finc) - jac(Ninc - n, Finc - f)
    mid = (lax.broadcasted_iota(jnp.int32, (1, K), 1).astype(jnp.float32)
           + 0.5) * (1.0 / K)
    losses = jnp.sum(dJ * mid, axis=1, keepdims=True)
    present = (G > 0).astype(jnp.float32)
    cnt = jnp.sum(present)
    tot = jnp.sum(losses * present)
    val = jnp.where(cnt > 0, tot / cnt, jnp.float32(0.0))
    out_ref[...] = jnp.reshape(val, (1, 1))


def kernel(outputs, labels):
    x = outputs.reshape(B, C, HW)
    lab = labels.reshape(B, HW).astype(jnp.int32)
    hists = _sc_hist(x, lab)
    hists = hists.reshape(NW, 2 * C, K)
    out = pl.pallas_call(
        _tc_finish_kernel,
        out_shape=jax.ShapeDtypeStruct((1, 1), jnp.float32),
    )(hists)
    return out.reshape(())
